# Initial kernel scaffold; baseline (speedup 1.0000x reference)
import jax
import jax.numpy as jnp
from jax.experimental import pallas as pl


def kernel(desc1, desc2):
    raise NotImplementedError("write your pallas kernel here")


# speedup vs baseline: 1.3263x; 1.3263x over previous
import jax
import jax.numpy as jnp
from jax.experimental import pallas as pl
from jax.experimental.pallas import tpu as pltpu

N = 8192
K = 128
BI = 1024
BJ = 2048


def _matcher_kernel(a_ref, bt_ref, a2_ref, b2_ref, dist_ref, idx_ref,
                    minv_ref, mini_ref):
    j = pl.program_id(1)
    nj = pl.num_programs(1)

    ab = jax.lax.dot_general(
        a_ref[...], bt_ref[...], (((1,), (0,)), ((), ())),
        preferred_element_type=jnp.float32)
    d2 = jnp.maximum((a2_ref[...] + b2_ref[...]) - 2.0 * ab, 0.0)

    bmin = jnp.min(d2, axis=1, keepdims=True)
    iota = jax.lax.broadcasted_iota(jnp.int32, d2.shape, 1)
    barg = jnp.min(jnp.where(d2 == bmin, iota, jnp.int32(2**30)),
                   axis=1, keepdims=True) + j * BJ

    @pl.when(j == 0)
    def _init():
        minv_ref[...] = bmin
        mini_ref[...] = barg

    @pl.when(j > 0)
    def _acc():
        prev = minv_ref[...]
        better = bmin < prev
        minv_ref[...] = jnp.where(better, bmin, prev)
        mini_ref[...] = jnp.where(better, barg, mini_ref[...])

    @pl.when(j == nj - 1)
    def _fin():
        dist_ref[...] = jnp.sqrt(minv_ref[...])
        idx_ref[...] = mini_ref[...]


def _match(desc1, bt, a2, b2t):
    return pl.pallas_call(
        _matcher_kernel,
        grid=(N // BI, N // BJ),
        in_specs=[
            pl.BlockSpec((BI, K), lambda i, j: (i, 0)),
            pl.BlockSpec((K, BJ), lambda i, j: (0, j)),
            pl.BlockSpec((BI, 1), lambda i, j: (i, 0)),
            pl.BlockSpec((1, BJ), lambda i, j: (0, j)),
        ],
        out_specs=[
            pl.BlockSpec((BI, 1), lambda i, j: (i, 0)),
            pl.BlockSpec((BI, 1), lambda i, j: (i, 0)),
        ],
        out_shape=[
            jax.ShapeDtypeStruct((N, 1), jnp.float32),
            jax.ShapeDtypeStruct((N, 1), jnp.int32),
        ],
        scratch_shapes=[
            pltpu.VMEM((BI, 1), jnp.float32),
            pltpu.VMEM((BI, 1), jnp.int32),
        ],
    )(desc1, bt, a2, b2t)


def kernel(desc1, desc2):
    a2 = jnp.sum(desc1 * desc1, axis=1, keepdims=True)
    b2t = jnp.sum(desc2 * desc2, axis=1, keepdims=True).T
    dists, idx2 = _match(desc1, desc2.T, a2, b2t)
    idx1 = jnp.arange(0, N, dtype=jnp.int32).reshape(-1, 1)
    return dists, jnp.concatenate([idx1, idx2], axis=1)


# speedup vs baseline: 2.1243x; 1.6017x over previous
import jax
import jax.numpy as jnp
from jax.experimental import pallas as pl
from jax.experimental.pallas import tpu as pltpu

N = 8192
K = 128
BI = 1024
BJ = 2048
G = BJ // 128
BIG = 2**30


def _matcher_kernel(a_ref, bt_ref, a2_ref, b2_ref, dist_ref, idx_ref,
                    rv_ref, rj_ref):
    j = pl.program_id(1)
    nj = pl.num_programs(1)

    nab = jax.lax.dot_general(
        a_ref[...], bt_ref[...], (((1,), (0,)), ((), ())),
        preferred_element_type=jnp.float32)
    b2 = b2_ref[...]

    val = b2[:, 0:128] + nab[:, 0:128]
    cid = jnp.zeros((BI, 128), jnp.int32)
    for g in range(1, G):
        s = b2[:, g * 128:(g + 1) * 128] + nab[:, g * 128:(g + 1) * 128]
        lt = s < val
        val = jnp.where(lt, s, val)
        cid = jnp.where(lt, jnp.int32(g), cid)
    cid = cid + j * G

    @pl.when(j == 0)
    def _init():
        rv_ref[...] = val
        rj_ref[...] = cid

    @pl.when(j > 0)
    def _acc():
        prev = rv_ref[...]
        lt = val < prev
        rv_ref[...] = jnp.where(lt, val, prev)
        rj_ref[...] = jnp.where(lt, cid, rj_ref[...])

    @pl.when(j == nj - 1)
    def _fin():
        fv = rv_ref[...]
        bmin = jnp.min(fv, axis=1, keepdims=True)
        lane = jax.lax.broadcasted_iota(jnp.int32, (BI, 128), 1)
        jfull = rj_ref[...] * 128 + lane
        idx_ref[...] = jnp.min(jnp.where(fv == bmin, jfull, BIG),
                               axis=1, keepdims=True)
        dist_ref[...] = jnp.sqrt(jnp.maximum(a2_ref[...] + bmin, 0.0))


def _match(desc1, bt, a2, b2t):
    return pl.pallas_call(
        _matcher_kernel,
        grid=(N // BI, N // BJ),
        in_specs=[
            pl.BlockSpec((BI, K), lambda i, j: (i, 0)),
            pl.BlockSpec((K, BJ), lambda i, j: (0, j)),
            pl.BlockSpec((BI, 1), lambda i, j: (i, 0)),
            pl.BlockSpec((1, BJ), lambda i, j: (0, j)),
        ],
        out_specs=[
            pl.BlockSpec((BI, 1), lambda i, j: (i, 0)),
            pl.BlockSpec((BI, 1), lambda i, j: (i, 0)),
        ],
        out_shape=[
            jax.ShapeDtypeStruct((N, 1), jnp.float32),
            jax.ShapeDtypeStruct((N, 1), jnp.int32),
        ],
        scratch_shapes=[
            pltpu.VMEM((BI, 128), jnp.float32),
            pltpu.VMEM((BI, 128), jnp.int32),
        ],
    )(desc1, bt, a2, b2t)


def kernel(desc1, desc2):
    a2 = jnp.sum(desc1 * desc1, axis=1, keepdims=True)
    b2t = jnp.sum(desc2 * desc2, axis=1, keepdims=True).T
    dists, idx2 = _match(desc1, -2.0 * desc2.T, a2, b2t)
    idx1 = jnp.arange(0, N, dtype=jnp.int32).reshape(-1, 1)
    return dists, jnp.concatenate([idx1, idx2], axis=1)


# speedup vs baseline: 2.4692x; 1.1624x over previous
import jax
import jax.numpy as jnp
from jax.experimental import pallas as pl
from jax.experimental.pallas import tpu as pltpu

N = 8192
K = 128
BI = 1024
BJ = 8192
G = BJ // 128
BIG = 2**30


def _matcher_kernel(a_ref, bt_ref, a2_ref, b2_ref, dist_ref, idx_ref,
                    rv_ref, rj_ref):
    j = pl.program_id(1)
    nj = pl.num_programs(1)

    nab = jax.lax.dot_general(
        a_ref[...], bt_ref[...], (((1,), (0,)), ((), ())),
        preferred_element_type=jnp.float32)
    b2 = b2_ref[...]

    val = b2[:, 0:128] + nab[:, 0:128]
    cid = jnp.zeros((BI, 128), jnp.int32)
    for g in range(1, G):
        s = b2[:, g * 128:(g + 1) * 128] + nab[:, g * 128:(g + 1) * 128]
        lt = s < val
        val = jnp.where(lt, s, val)
        cid = jnp.where(lt, jnp.int32(g), cid)
    cid = cid + j * G

    @pl.when(j == 0)
    def _init():
        rv_ref[...] = val
        rj_ref[...] = cid

    @pl.when(j > 0)
    def _acc():
        prev = rv_ref[...]
        lt = val < prev
        rv_ref[...] = jnp.where(lt, val, prev)
        rj_ref[...] = jnp.where(lt, cid, rj_ref[...])

    @pl.when(j == nj - 1)
    def _fin():
        fv = rv_ref[...]
        bmin = jnp.min(fv, axis=1, keepdims=True)
        lane = jax.lax.broadcasted_iota(jnp.int32, (BI, 128), 1)
        jfull = rj_ref[...] * 128 + lane
        idx_ref[...] = jnp.min(jnp.where(fv == bmin, jfull, BIG),
                               axis=1, keepdims=True)
        dist_ref[...] = jnp.sqrt(jnp.maximum(a2_ref[...] + bmin, 0.0))


def _match(desc1, bt, a2, b2t):
    return pl.pallas_call(
        _matcher_kernel,
        grid=(N // BI, N // BJ),
        in_specs=[
            pl.BlockSpec((BI, K), lambda i, j: (i, 0)),
            pl.BlockSpec((K, BJ), lambda i, j: (0, j)),
            pl.BlockSpec((BI, 1), lambda i, j: (i, 0)),
            pl.BlockSpec((1, BJ), lambda i, j: (0, j)),
        ],
        out_specs=[
            pl.BlockSpec((BI, 1), lambda i, j: (i, 0)),
            pl.BlockSpec((BI, 1), lambda i, j: (i, 0)),
        ],
        out_shape=[
            jax.ShapeDtypeStruct((N, 1), jnp.float32),
            jax.ShapeDtypeStruct((N, 1), jnp.int32),
        ],
        scratch_shapes=[
            pltpu.VMEM((BI, 128), jnp.float32),
            pltpu.VMEM((BI, 128), jnp.int32),
        ],
    )(desc1, bt, a2, b2t)


def kernel(desc1, desc2):
    a2 = jnp.sum(desc1 * desc1, axis=1, keepdims=True)
    b2t = jnp.sum(desc2 * desc2, axis=1, keepdims=True).T
    dists, idx2 = _match(desc1, -2.0 * desc2.T, a2, b2t)
    idx1 = jnp.arange(0, N, dtype=jnp.int32).reshape(-1, 1)
    return dists, jnp.concatenate([idx1, idx2], axis=1)


# speedup vs baseline: 2.5739x; 1.0424x over previous
import jax
import jax.numpy as jnp
from jax.experimental import pallas as pl
from jax.experimental.pallas import tpu as pltpu

N = 8192
K = 128
BI = 1024
BJ = 8192
G = BJ // 128
BIG = 2**30


def _matcher_kernel(a_ref, bt_ref, a2_ref, b2_ref, dist_ref, idx_ref,
                    rv_ref, rj_ref):
    j = pl.program_id(1)
    nj = pl.num_programs(1)

    nab = jax.lax.dot_general(
        a_ref[...], bt_ref[...], (((1,), (1,)), ((), ())),
        preferred_element_type=jnp.float32)
    b2 = b2_ref[...]

    val = b2[:, 0:128] + nab[:, 0:128]
    cid = jnp.zeros((BI, 128), jnp.int32)
    for g in range(1, G):
        s = b2[:, g * 128:(g + 1) * 128] + nab[:, g * 128:(g + 1) * 128]
        lt = s < val
        val = jnp.where(lt, s, val)
        cid = jnp.where(lt, jnp.int32(g), cid)
    cid = cid + j * G

    @pl.when(j == 0)
    def _init():
        rv_ref[...] = val
        rj_ref[...] = cid

    @pl.when(j > 0)
    def _acc():
        prev = rv_ref[...]
        lt = val < prev
        rv_ref[...] = jnp.where(lt, val, prev)
        rj_ref[...] = jnp.where(lt, cid, rj_ref[...])

    @pl.when(j == nj - 1)
    def _fin():
        fv = rv_ref[...]
        bmin = jnp.min(fv, axis=1, keepdims=True)
        lane = jax.lax.broadcasted_iota(jnp.int32, (BI, 128), 1)
        jfull = rj_ref[...] * 128 + lane
        idx_ref[...] = jnp.min(jnp.where(fv == bmin, jfull, BIG),
                               axis=1, keepdims=True)
        dist_ref[...] = jnp.sqrt(jnp.maximum(a2_ref[...] + bmin, 0.0))


def _match(am2, b, a2, b2t):
    return pl.pallas_call(
        _matcher_kernel,
        grid=(N // BI, N // BJ),
        in_specs=[
            pl.BlockSpec((BI, K), lambda i, j: (i, 0)),
            pl.BlockSpec((BJ, K), lambda i, j: (j, 0)),
            pl.BlockSpec((BI, 1), lambda i, j: (i, 0)),
            pl.BlockSpec((1, BJ), lambda i, j: (0, j)),
        ],
        out_specs=[
            pl.BlockSpec((BI, 1), lambda i, j: (i, 0)),
            pl.BlockSpec((BI, 1), lambda i, j: (i, 0)),
        ],
        out_shape=[
            jax.ShapeDtypeStruct((N, 1), jnp.float32),
            jax.ShapeDtypeStruct((N, 1), jnp.int32),
        ],
        scratch_shapes=[
            pltpu.VMEM((BI, 128), jnp.float32),
            pltpu.VMEM((BI, 128), jnp.int32),
        ],
    )(am2, b, a2, b2t)


def kernel(desc1, desc2):
    a2 = jnp.sum(desc1 * desc1, axis=1, keepdims=True)
    b2t = jnp.sum(desc2 * desc2, axis=1, keepdims=True).T
    dists, idx2 = _match(-2.0 * desc1, desc2, a2, b2t)
    idx1 = jnp.arange(0, N, dtype=jnp.int32).reshape(-1, 1)
    return dists, jnp.concatenate([idx1, idx2], axis=1)


# speedup vs baseline: 2.8344x; 1.1012x over previous
import jax
import jax.numpy as jnp
from jax.experimental import pallas as pl
from jax.experimental.pallas import tpu as pltpu

N = 8192
K = 128
BI = 1024
BJ = 8192
G = BJ // 128
BIG = 2**30


def _matcher_kernel(a_ref, b_ref, b2_ref, dist_ref, idx_ref,
                    rv_ref, rj_ref):
    j = pl.program_id(1)
    nj = pl.num_programs(1)

    a = a_ref[...]
    nab = jax.lax.dot_general(
        a * -2.0, b_ref[...], (((1,), (1,)), ((), ())),
        preferred_element_type=jnp.float32)
    b2 = b2_ref[...]

    val = b2[:, 0:128] + nab[:, 0:128]
    cid = jnp.zeros((BI, 128), jnp.int32)
    for g in range(1, G):
        s = b2[:, g * 128:(g + 1) * 128] + nab[:, g * 128:(g + 1) * 128]
        lt = s < val
        val = jnp.where(lt, s, val)
        cid = jnp.where(lt, jnp.int32(g), cid)
    cid = cid + j * G

    @pl.when(j == 0)
    def _init():
        rv_ref[...] = val
        rj_ref[...] = cid

    @pl.when(j > 0)
    def _acc():
        prev = rv_ref[...]
        lt = val < prev
        rv_ref[...] = jnp.where(lt, val, prev)
        rj_ref[...] = jnp.where(lt, cid, rj_ref[...])

    @pl.when(j == nj - 1)
    def _fin():
        fv = rv_ref[...]
        bmin = jnp.min(fv, axis=1, keepdims=True)
        lane = jax.lax.broadcasted_iota(jnp.int32, (BI, 128), 1)
        jfull = rj_ref[...] * 128 + lane
        idx_ref[...] = jnp.min(jnp.where(fv == bmin, jfull, BIG),
                               axis=1, keepdims=True)
        a2 = jnp.sum(a * a, axis=1, keepdims=True)
        dist_ref[...] = jnp.sqrt(jnp.maximum(a2 + bmin, 0.0))


def _match(desc1, desc2, b2t):
    return pl.pallas_call(
        _matcher_kernel,
        grid=(N // BI, N // BJ),
        in_specs=[
            pl.BlockSpec((BI, K), lambda i, j: (i, 0)),
            pl.BlockSpec((BJ, K), lambda i, j: (j, 0)),
            pl.BlockSpec((1, BJ), lambda i, j: (0, j)),
        ],
        out_specs=[
            pl.BlockSpec((BI, 1), lambda i, j: (i, 0)),
            pl.BlockSpec((BI, 1), lambda i, j: (i, 0)),
        ],
        out_shape=[
            jax.ShapeDtypeStruct((N, 1), jnp.float32),
            jax.ShapeDtypeStruct((N, 1), jnp.int32),
        ],
        scratch_shapes=[
            pltpu.VMEM((BI, 128), jnp.float32),
            pltpu.VMEM((BI, 128), jnp.int32),
        ],
    )(desc1, desc2, b2t)


def kernel(desc1, desc2):
    b2t = jnp.sum(desc2 * desc2, axis=1, keepdims=True).T
    dists, idx2 = _match(desc1, desc2, b2t)
    idx1 = jnp.arange(0, N, dtype=jnp.int32).reshape(-1, 1)
    return dists, jnp.concatenate([idx1, idx2], axis=1)
